# baseline (speedup 1.0000x reference)
import jax
import jax.numpy as jnp
from jax.experimental import pallas as pl
from jax.experimental.pallas import tpu as pltpu

_DT_RANK = 160
_D_STATE = 32

_DBLK1 = 512
_DBLK2 = 512


def _stage1_body(x_ref, wssm_ref, wmlp_ref, cs1_ref, cs23_ref, cw_ref,
                 cb_ref, xpt_ref, u_ref, res_ref, dbl_ref):
    i = pl.program_id(1)
    x = x_ref[...]
    xssm = jnp.dot(x, wssm_ref[...], preferred_element_type=jnp.float32)
    res = jax.nn.silu(jnp.dot(x, wmlp_ref[...],
                              preferred_element_type=jnp.float32))
    cw = cw_ref[...]
    conv = (cs1_ref[...] * cw[0:1] + cs23_ref[:128] * cw[1:2]
            + cs23_ref[128:] * cw[2:3] + xssm * cw[3:4] + cb_ref[...])
    u = jax.nn.silu(conv)
    u_ref[...] = u.astype(jnp.bfloat16)
    res_ref[...] = res.astype(jnp.bfloat16)
    contrib = jax.lax.dot_general(
        u, xpt_ref[...], (((1,), (1,)), ((), ())),
        preferred_element_type=jnp.float32)

    @pl.when(i == 0)
    def _():
        dbl_ref[0] = contrib

    @pl.when(i != 0)
    def _():
        dbl_ref[0] += contrib


def _stage2_body(dblp_ref, dtw_ref, dtb_ref, alogt_ref,
                 ssmt_ref, u_ref, res_ref, d_ref, outw_ref, out_ref):
    i = pl.program_id(1)
    dbl = dblp_ref[0] + dblp_ref[1]
    dtin = dbl[:, :_DT_RANK]
    bm = dbl[:, _DT_RANK:_DT_RANK + _D_STATE]
    cm = dbl[:, _DT_RANK + _D_STATE:]
    dt = jax.nn.softplus(
        jnp.dot(dtin, dtw_ref[...],
                preferred_element_type=jnp.float32) + dtb_ref[...])
    a = -jnp.exp(alogt_ref[...])
    dA = jnp.exp(dt[:, None, :] * a[None, :, :])
    z = dA * ssmt_ref[...] * cm[:, :, None]
    y1 = jnp.sum(z, axis=1)
    bc = jnp.sum(bm * cm, axis=1, keepdims=True)
    u = u_ref[...].astype(jnp.float32)
    y = y1 + dt * u * bc + d_ref[...] * u
    g = y * res_ref[...].astype(jnp.float32)
    contrib = jnp.dot(g, outw_ref[...], preferred_element_type=jnp.float32)

    @pl.when(i == 0)
    def _():
        out_ref[0] = contrib

    @pl.when(i != 0)
    def _():
        out_ref[0] += contrib


def kernel(x, conv_states, ssm_state, w_in_ssm, w_in_mlp, conv_w, conv_b,
           A_log, x_proj_w, dt_proj_w, dt_proj_b, D, out_proj_w):
    B, DM = x.shape[2], x.shape[3]
    DI = w_in_ssm.shape[1]
    x2 = x.reshape(B, DM)
    cb = conv_b.reshape(1, DI)
    xpt = x_proj_w.T
    csf = conv_states.reshape(4 * B, DI)

    n1 = DI // _DBLK1 // 2
    u, res, dbl_parts = pl.pallas_call(
        _stage1_body,
        grid=(2, n1),
        in_specs=[
            pl.BlockSpec((B, DM), lambda c, i: (0, 0)),
            pl.BlockSpec((DM, _DBLK1), lambda c, i: (0, c * n1 + i)),
            pl.BlockSpec((DM, _DBLK1), lambda c, i: (0, c * n1 + i)),
            pl.BlockSpec((B, _DBLK1), lambda c, i: (1, c * n1 + i)),
            pl.BlockSpec((2 * B, _DBLK1), lambda c, i: (1, c * n1 + i)),
            pl.BlockSpec((4, _DBLK1), lambda c, i: (0, c * n1 + i)),
            pl.BlockSpec((1, _DBLK1), lambda c, i: (0, c * n1 + i)),
            pl.BlockSpec((_DT_RANK + 2 * _D_STATE, _DBLK1),
                         lambda c, i: (0, c * n1 + i)),
        ],
        out_specs=[
            pl.BlockSpec((B, _DBLK1), lambda c, i: (0, c * n1 + i)),
            pl.BlockSpec((B, _DBLK1), lambda c, i: (0, c * n1 + i)),
            pl.BlockSpec((1, B, _DT_RANK + 2 * _D_STATE),
                         lambda c, i: (c, 0, 0)),
        ],
        out_shape=[
            jax.ShapeDtypeStruct((B, DI), jnp.bfloat16),
            jax.ShapeDtypeStruct((B, DI), jnp.bfloat16),
            jax.ShapeDtypeStruct((2, B, _DT_RANK + 2 * _D_STATE),
                                 jnp.float32),
        ],
        compiler_params=pltpu.CompilerParams(
            dimension_semantics=("parallel", "arbitrary"),
            vmem_limit_bytes=56 * 1024 * 1024,
        ),
    )(x2, w_in_ssm, w_in_mlp, csf, csf, conv_w, cb, xpt)

    ssmt = ssm_state.transpose(0, 2, 1)
    alogt = A_log.T
    dtb = dt_proj_b.reshape(1, DI)
    d2 = D.reshape(1, DI)

    n2 = DI // _DBLK2 // 2
    out_parts = pl.pallas_call(
        _stage2_body,
        grid=(2, n2),
        in_specs=[
            pl.BlockSpec((2, B, _DT_RANK + 2 * _D_STATE),
                         lambda c, i: (0, 0, 0)),
            pl.BlockSpec((_DT_RANK, _DBLK2), lambda c, i: (0, c * n2 + i)),
            pl.BlockSpec((1, _DBLK2), lambda c, i: (0, c * n2 + i)),
            pl.BlockSpec((_D_STATE, _DBLK2), lambda c, i: (0, c * n2 + i)),
            pl.BlockSpec((B, _D_STATE, _DBLK2),
                         lambda c, i: (0, 0, c * n2 + i)),
            pl.BlockSpec((B, _DBLK2), lambda c, i: (0, c * n2 + i)),
            pl.BlockSpec((B, _DBLK2), lambda c, i: (0, c * n2 + i)),
            pl.BlockSpec((1, _DBLK2), lambda c, i: (0, c * n2 + i)),
            pl.BlockSpec((_DBLK2, DM), lambda c, i: (c * n2 + i, 0)),
        ],
        out_specs=pl.BlockSpec((1, B, DM), lambda c, i: (c, 0, 0)),
        out_shape=jax.ShapeDtypeStruct((2, B, DM), jnp.float32),
        compiler_params=pltpu.CompilerParams(
            dimension_semantics=("parallel", "arbitrary"),
            vmem_limit_bytes=56 * 1024 * 1024,
        ),
    )(dbl_parts, dt_proj_w, dtb, alogt, ssmt, u, res, d2, out_proj_w)

    out = out_parts[0] + out_parts[1]
    return out.reshape(1, 1, B, DM)


# speedup vs baseline: 1.0262x; 1.0262x over previous
import jax
import jax.numpy as jnp
from jax.experimental import pallas as pl
from jax.experimental.pallas import tpu as pltpu

_DT_RANK = 160
_D_STATE = 32

_DBLK1 = 512
_DBLK2 = 512


def _stage1_body(x_ref, wssm_ref, wmlp_ref, cs1_ref, cs23_ref, cw_ref,
                 cb_ref, xpt_ref, u_ref, res_ref, dbl_ref):
    i = pl.program_id(1)
    x = x_ref[...]
    xssm = jnp.dot(x, wssm_ref[...], preferred_element_type=jnp.float32)
    res = jax.nn.silu(jnp.dot(x, wmlp_ref[...],
                              preferred_element_type=jnp.float32))
    cw = cw_ref[...]
    conv = (cs1_ref[...] * cw[0:1] + cs23_ref[:128] * cw[1:2]
            + cs23_ref[128:] * cw[2:3] + xssm * cw[3:4] + cb_ref[...])
    u = jax.nn.silu(conv)
    u_ref[...] = u
    res_ref[...] = res
    contrib = jax.lax.dot_general(
        u, xpt_ref[...], (((1,), (1,)), ((), ())),
        preferred_element_type=jnp.float32)

    @pl.when(i == 0)
    def _():
        dbl_ref[0] = contrib

    @pl.when(i != 0)
    def _():
        dbl_ref[0] += contrib


def _stage2_body(dblp_ref, dtw_ref, dtb_ref, alogt_ref,
                 ssmt_ref, u_ref, res_ref, d_ref, outw_ref, out_ref):
    i = pl.program_id(1)
    dbl = dblp_ref[0] + dblp_ref[1]
    dtin = dbl[:, :_DT_RANK]
    bm = dbl[:, _DT_RANK:_DT_RANK + _D_STATE]
    cm = dbl[:, _DT_RANK + _D_STATE:]
    dt = jax.nn.softplus(
        jnp.dot(dtin, dtw_ref[...],
                preferred_element_type=jnp.float32) + dtb_ref[...])
    a = -jnp.exp(alogt_ref[...])
    dA = jnp.exp(dt[:, None, :] * a[None, :, :])
    z = dA * ssmt_ref[...] * cm[:, :, None]
    y1 = jnp.sum(z, axis=1)
    bc = jnp.sum(bm * cm, axis=1, keepdims=True)
    u = u_ref[...]
    y = y1 + dt * u * bc + d_ref[...] * u
    g = y * res_ref[...]
    contrib = jnp.dot(g, outw_ref[...], preferred_element_type=jnp.float32)

    @pl.when(i == 0)
    def _():
        out_ref[0] = contrib

    @pl.when(i != 0)
    def _():
        out_ref[0] += contrib


def kernel(x, conv_states, ssm_state, w_in_ssm, w_in_mlp, conv_w, conv_b,
           A_log, x_proj_w, dt_proj_w, dt_proj_b, D, out_proj_w):
    B, DM = x.shape[2], x.shape[3]
    DI = w_in_ssm.shape[1]
    x2 = x.reshape(B, DM)
    cb = conv_b.reshape(1, DI)
    xpt = x_proj_w.T
    csf = conv_states.reshape(4 * B, DI)

    n1 = DI // _DBLK1 // 2
    u, res, dbl_parts = pl.pallas_call(
        _stage1_body,
        grid=(2, n1),
        in_specs=[
            pl.BlockSpec((B, DM), lambda c, i: (0, 0)),
            pl.BlockSpec((DM, _DBLK1), lambda c, i: (0, c * n1 + i)),
            pl.BlockSpec((DM, _DBLK1), lambda c, i: (0, c * n1 + i)),
            pl.BlockSpec((B, _DBLK1), lambda c, i: (1, c * n1 + i)),
            pl.BlockSpec((2 * B, _DBLK1), lambda c, i: (1, c * n1 + i)),
            pl.BlockSpec((4, _DBLK1), lambda c, i: (0, c * n1 + i)),
            pl.BlockSpec((1, _DBLK1), lambda c, i: (0, c * n1 + i)),
            pl.BlockSpec((_DT_RANK + 2 * _D_STATE, _DBLK1),
                         lambda c, i: (0, c * n1 + i)),
        ],
        out_specs=[
            pl.BlockSpec((B, _DBLK1), lambda c, i: (0, c * n1 + i)),
            pl.BlockSpec((B, _DBLK1), lambda c, i: (0, c * n1 + i)),
            pl.BlockSpec((1, B, _DT_RANK + 2 * _D_STATE),
                         lambda c, i: (c, 0, 0)),
        ],
        out_shape=[
            jax.ShapeDtypeStruct((B, DI), jnp.float32),
            jax.ShapeDtypeStruct((B, DI), jnp.float32),
            jax.ShapeDtypeStruct((2, B, _DT_RANK + 2 * _D_STATE),
                                 jnp.float32),
        ],
        compiler_params=pltpu.CompilerParams(
            dimension_semantics=("parallel", "arbitrary"),
            vmem_limit_bytes=56 * 1024 * 1024,
        ),
    )(x2, w_in_ssm, w_in_mlp, csf, csf, conv_w, cb, xpt)

    ssmt = ssm_state.transpose(0, 2, 1)
    alogt = A_log.T
    dtb = dt_proj_b.reshape(1, DI)
    d2 = D.reshape(1, DI)

    n2 = DI // _DBLK2 // 2
    out_parts = pl.pallas_call(
        _stage2_body,
        grid=(2, n2),
        in_specs=[
            pl.BlockSpec((2, B, _DT_RANK + 2 * _D_STATE),
                         lambda c, i: (0, 0, 0)),
            pl.BlockSpec((_DT_RANK, _DBLK2), lambda c, i: (0, c * n2 + i)),
            pl.BlockSpec((1, _DBLK2), lambda c, i: (0, c * n2 + i)),
            pl.BlockSpec((_D_STATE, _DBLK2), lambda c, i: (0, c * n2 + i)),
            pl.BlockSpec((B, _D_STATE, _DBLK2),
                         lambda c, i: (0, 0, c * n2 + i)),
            pl.BlockSpec((B, _DBLK2), lambda c, i: (0, c * n2 + i)),
            pl.BlockSpec((B, _DBLK2), lambda c, i: (0, c * n2 + i)),
            pl.BlockSpec((1, _DBLK2), lambda c, i: (0, c * n2 + i)),
            pl.BlockSpec((_DBLK2, DM), lambda c, i: (c * n2 + i, 0)),
        ],
        out_specs=pl.BlockSpec((1, B, DM), lambda c, i: (c, 0, 0)),
        out_shape=jax.ShapeDtypeStruct((2, B, DM), jnp.float32),
        compiler_params=pltpu.CompilerParams(
            dimension_semantics=("parallel", "arbitrary"),
            vmem_limit_bytes=56 * 1024 * 1024,
        ),
    )(dbl_parts, dt_proj_w, dtb, alogt, ssmt, u, res, d2, out_proj_w)

    out = out_parts[0] + out_parts[1]
    return out.reshape(1, 1, B, DM)


# speedup vs baseline: 1.0627x; 1.0355x over previous
import jax
import jax.numpy as jnp
from jax.experimental import pallas as pl
from jax.experimental.pallas import tpu as pltpu

_DT_RANK = 160
_D_STATE = 32

_DBLK1 = 512
_DBLK2 = 512


def _stage1_body(x_ref, wssm_ref, wmlp_ref, cs1_ref, cs23_ref, cw_ref,
                 cb_ref, xpt_ref, u_ref, res_ref, dbl_ref):
    i = pl.program_id(0)
    x = x_ref[...]
    xssm = jnp.dot(x, wssm_ref[...], preferred_element_type=jnp.float32)
    res = jax.nn.silu(jnp.dot(x, wmlp_ref[...],
                              preferred_element_type=jnp.float32))
    cw = cw_ref[...]
    conv = (cs1_ref[...] * cw[0:1] + cs23_ref[:128] * cw[1:2]
            + cs23_ref[128:] * cw[2:3] + xssm * cw[3:4] + cb_ref[...])
    u = jax.nn.silu(conv)
    u_ref[...] = u
    res_ref[...] = res
    contrib = jax.lax.dot_general(
        u, xpt_ref[...], (((1,), (1,)), ((), ())),
        preferred_element_type=jnp.float32)

    @pl.when(i == 0)
    def _():
        dbl_ref[...] = contrib

    @pl.when(i != 0)
    def _():
        dbl_ref[...] += contrib


def _stage2_body(dblp_ref, dtw_ref, dtb_ref, alogt_ref,
                 ssmt_ref, u_ref, res_ref, d_ref, outw_ref, out_ref):
    i = pl.program_id(0)
    dbl = dblp_ref[...]
    dtin = dbl[:, :_DT_RANK]
    bm = dbl[:, _DT_RANK:_DT_RANK + _D_STATE]
    cm = dbl[:, _DT_RANK + _D_STATE:]
    dt = jax.nn.softplus(
        jnp.dot(dtin, dtw_ref[...],
                preferred_element_type=jnp.float32) + dtb_ref[...])
    a = -jnp.exp(alogt_ref[...])
    dA = jnp.exp(dt[:, None, :] * a[None, :, :])
    z = dA * ssmt_ref[...] * cm[:, :, None]
    y1 = jnp.sum(z, axis=1)
    bc = jnp.sum(bm * cm, axis=1, keepdims=True)
    u = u_ref[...]
    y = y1 + dt * u * bc + d_ref[...] * u
    g = y * res_ref[...]
    contrib = jnp.dot(g, outw_ref[...], preferred_element_type=jnp.float32)

    @pl.when(i == 0)
    def _():
        out_ref[...] = contrib

    @pl.when(i != 0)
    def _():
        out_ref[...] += contrib


def kernel(x, conv_states, ssm_state, w_in_ssm, w_in_mlp, conv_w, conv_b,
           A_log, x_proj_w, dt_proj_w, dt_proj_b, D, out_proj_w):
    B, DM = x.shape[2], x.shape[3]
    DI = w_in_ssm.shape[1]
    x2 = x.reshape(B, DM)
    cb = conv_b.reshape(1, DI)
    xpt = x_proj_w.T
    csf = conv_states.reshape(4 * B, DI)

    n1 = DI // _DBLK1
    u, res, dbl = pl.pallas_call(
        _stage1_body,
        grid=(n1,),
        in_specs=[
            pl.BlockSpec((B, DM), lambda i: (0, 0)),
            pl.BlockSpec((DM, _DBLK1), lambda i: (0, i)),
            pl.BlockSpec((DM, _DBLK1), lambda i: (0, i)),
            pl.BlockSpec((B, _DBLK1), lambda i: (1, i)),
            pl.BlockSpec((2 * B, _DBLK1), lambda i: (1, i)),
            pl.BlockSpec((4, _DBLK1), lambda i: (0, i)),
            pl.BlockSpec((1, _DBLK1), lambda i: (0, i)),
            pl.BlockSpec((_DT_RANK + 2 * _D_STATE, _DBLK1),
                         lambda i: (0, i)),
        ],
        out_specs=[
            pl.BlockSpec((B, _DBLK1), lambda i: (0, i)),
            pl.BlockSpec((B, _DBLK1), lambda i: (0, i)),
            pl.BlockSpec((B, _DT_RANK + 2 * _D_STATE), lambda i: (0, 0)),
        ],
        out_shape=[
            jax.ShapeDtypeStruct((B, DI), jnp.float32),
            jax.ShapeDtypeStruct((B, DI), jnp.float32),
            jax.ShapeDtypeStruct((B, _DT_RANK + 2 * _D_STATE), jnp.float32),
        ],
        compiler_params=pltpu.CompilerParams(
            dimension_semantics=("arbitrary",),
            vmem_limit_bytes=56 * 1024 * 1024,
        ),
    )(x2, w_in_ssm, w_in_mlp, csf, csf, conv_w, cb, xpt)

    ssmt = ssm_state.transpose(0, 2, 1)
    alogt = A_log.T
    dtb = dt_proj_b.reshape(1, DI)
    d2 = D.reshape(1, DI)

    n2 = DI // _DBLK2
    out = pl.pallas_call(
        _stage2_body,
        grid=(n2,),
        in_specs=[
            pl.BlockSpec((B, _DT_RANK + 2 * _D_STATE), lambda i: (0, 0)),
            pl.BlockSpec((_DT_RANK, _DBLK2), lambda i: (0, i)),
            pl.BlockSpec((1, _DBLK2), lambda i: (0, i)),
            pl.BlockSpec((_D_STATE, _DBLK2), lambda i: (0, i)),
            pl.BlockSpec((B, _D_STATE, _DBLK2), lambda i: (0, 0, i)),
            pl.BlockSpec((B, _DBLK2), lambda i: (0, i)),
            pl.BlockSpec((B, _DBLK2), lambda i: (0, i)),
            pl.BlockSpec((1, _DBLK2), lambda i: (0, i)),
            pl.BlockSpec((_DBLK2, DM), lambda i: (i, 0)),
        ],
        out_specs=pl.BlockSpec((B, DM), lambda i: (0, 0)),
        out_shape=jax.ShapeDtypeStruct((B, DM), jnp.float32),
        compiler_params=pltpu.CompilerParams(
            dimension_semantics=("arbitrary",),
            vmem_limit_bytes=56 * 1024 * 1024,
        ),
    )(dbl, dt_proj_w, dtb, alogt, ssmt, u, res, d2, out_proj_w)

    return out.reshape(1, 1, B, DM)
